# baseline (speedup 1.0000x reference)
import functools

import jax
import jax.numpy as jnp
from jax import lax
from jax.experimental import pallas as pl
from jax.experimental.pallas import tpu as pltpu
from jax.experimental.pallas import tpu_sc as plsc

B = 8192
D = 768
L = 256
K = 8192

BB = 512
LANES = 128

NW = 32
EPW = B * D // NW
CH = 12288
NCH = EPW // CH


def _tc_body(h_ref, cc_ref, out_ref, c2_ref, ccb_ref):
    i = pl.program_id(0)

    @pl.when(i == 0)
    def _():
        cf = cc_ref[...]
        c2 = jnp.sum(cf * cf, axis=1)
        c2_ref[...] = (0.5 * c2).reshape(1, K).astype(jnp.bfloat16)
        ccb_ref[...] = cf.astype(jnp.bfloat16)

    h = h_ref[...]
    h2 = jnp.sum(h * h, axis=1)

    s = jax.lax.dot_general(
        h.astype(jnp.bfloat16), ccb_ref[...],
        (((1,), (1,)), ((), ())),
        preferred_element_type=jnp.float32,
    )

    sb = s.astype(jnp.bfloat16)
    m = jnp.full((BB, LANES), -jnp.inf, dtype=jnp.bfloat16)
    for t in range(K // LANES):
        sl = slice(t * LANES, (t + 1) * LANES)
        m = jnp.maximum(m, sb[:, sl] - c2_ref[0:1, sl])
    m_row = jnp.max(m.astype(jnp.float32), axis=1)

    d2 = h2 - 2.0 * m_row
    part = jnp.reshape(jnp.sum(d2) / (B * L), (1, 1))

    @pl.when(i == 0)
    def _():
        out_ref[...] = jnp.zeros((1, 1), jnp.float32)
    out_ref[...] += part


def _sc_recon_body(x_hbm, a_hbm, out_hbm, xb, ab, acc_ref, sem):
    wid = lax.axis_index("s") * 2 + lax.axis_index("c")
    base = wid * EPW

    def chunk(k, acc):
        pltpu.sync_copy(x_hbm.at[pl.ds(base + k * CH, CH)], xb)
        pltpu.sync_copy(a_hbm.at[pl.ds(base + k * CH, CH)], ab)

        def inner(j, acc):
            d = xb[pl.ds(j * 16, 16)] - ab[pl.ds(j * 16, 16)]
            return acc + d * d

        return lax.fori_loop(0, CH // 16, inner, acc)

    acc = lax.fori_loop(0, NCH, chunk, jnp.zeros((16,), jnp.float32))
    acc_ref[...] = acc
    pltpu.sync_copy(acc_ref, out_hbm.at[wid])


@functools.partial(
    pl.kernel,
    mesh=plsc.VectorSubcoreMesh(core_axis_name="c", subcore_axis_name="s"),
    out_type=jax.ShapeDtypeStruct((NW, 16), jnp.float32),
    scratch_types=[
        pltpu.VMEM((CH,), jnp.float32),
        pltpu.VMEM((CH,), jnp.float32),
        pltpu.VMEM((16,), jnp.float32),
        pltpu.SemaphoreType.DMA,
    ],
)
def _sc_recon(x_hbm, a_hbm, out_hbm, xb, ab, acc_ref, sem):
    _sc_recon_body(x_hbm, a_hbm, out_hbm, xb, ab, acc_ref, sem)


def kernel(x, h_x, a_x, cluster_centers):
    recon_parts = _sc_recon(x.reshape(-1), a_x.reshape(-1))
    clust = pl.pallas_call(
        _tc_body,
        grid=(B // BB,),
        in_specs=[
            pl.BlockSpec((BB, L), lambda i: (i, 0)),
            pl.BlockSpec((K, L), lambda i: (0, 0)),
        ],
        out_specs=pl.BlockSpec((1, 1), lambda i: (0, 0)),
        out_shape=jax.ShapeDtypeStruct((1, 1), jnp.float32),
        scratch_shapes=[pltpu.VMEM((1, K), jnp.bfloat16),
                        pltpu.VMEM((K, L), jnp.bfloat16)],
    )(h_x, cluster_centers)
    return clust[0, 0] + jnp.sum(recon_parts) / (B * D)


# speedup vs baseline: 2.9595x; 2.9595x over previous
import jax
import jax.numpy as jnp
from jax.experimental import pallas as pl
from jax.experimental.pallas import tpu as pltpu

B = 8192
D = 768
L = 256
K = 8192

BB = 512
KC = 2048
LANES = 128


def _loss_body(x_ref, a_ref, h_ref, cc_ref, out_ref, c2_ref, ccb_ref):
    i = pl.program_id(0)

    @pl.when(i == 0)
    def _():
        cf = cc_ref[...]
        c2 = jnp.sum(cf * cf, axis=1)
        c2_ref[...] = (0.5 * c2).reshape(1, K).astype(jnp.bfloat16)
        ccb_ref[...] = cf.astype(jnp.bfloat16)

    diff = x_ref[...] - a_ref[...]
    recon = jnp.sum(diff * diff)

    h = h_ref[...]
    h2 = jnp.sum(h * h, axis=1)
    hb = h.astype(jnp.bfloat16)

    m = jnp.full((BB, LANES), -jnp.inf, dtype=jnp.bfloat16)
    for kc in range(K // KC):
        s = jax.lax.dot_general(
            hb, ccb_ref[kc * KC:(kc + 1) * KC, :],
            (((1,), (1,)), ((), ())),
            preferred_element_type=jnp.float32,
        )
        sb = s.astype(jnp.bfloat16)
        for t in range(KC // LANES):
            j = kc * KC + t * LANES
            m = jnp.maximum(m, sb[:, t * LANES:(t + 1) * LANES]
                            - c2_ref[0:1, j:j + LANES])
    m_row = jnp.max(m.astype(jnp.float32), axis=1)

    d2 = h2 - 2.0 * m_row
    part = jnp.reshape(recon / (B * D) + jnp.sum(d2) / (B * L), (1, 1))

    @pl.when(i == 0)
    def _():
        out_ref[...] = jnp.zeros((1, 1), jnp.float32)
    out_ref[...] += part


def kernel(x, h_x, a_x, cluster_centers):
    out = pl.pallas_call(
        _loss_body,
        grid=(B // BB,),
        in_specs=[
            pl.BlockSpec((BB, D), lambda i: (i, 0)),
            pl.BlockSpec((BB, D), lambda i: (i, 0)),
            pl.BlockSpec((BB, L), lambda i: (i, 0)),
            pl.BlockSpec((K, L), lambda i: (0, 0)),
        ],
        out_specs=pl.BlockSpec((1, 1), lambda i: (0, 0)),
        out_shape=jax.ShapeDtypeStruct((1, 1), jnp.float32),
        scratch_shapes=[pltpu.VMEM((1, K), jnp.bfloat16),
                        pltpu.VMEM((K, L), jnp.bfloat16)],
    )(x, a_x, h_x, cluster_centers)
    return out[0, 0]
